# baseline (speedup 1.0000x reference)
import jax
import jax.numpy as jnp
from jax.experimental import pallas as pl

N = 10000
M = 5000
E = 320000
D = 128
TVW = 144
TEW = 272
A1W = 144

TILE_N = 1000
TILE_M = 1000
TILE_E = 2000


def _ln(x, eps=1e-5):
    mu = jnp.mean(x, axis=-1, keepdims=True)
    var = jnp.var(x, axis=-1, keepdims=True)
    return (x - mu) / jnp.sqrt(var + eps)


def _mlp_body(x, W1, b1, W2, b2):
    h = jax.nn.relu(_ln(x @ W1 + b1))
    return h @ W2 + b2


def _stage_a_kernel(v_ref, W1n_ref, b1n_ref, W2n_ref, b2n_ref, W1et_ref,
                    nm_ref, tv_ref):
    x = v_ref[...]
    nm_ref[...] = _mlp_body(_ln(x), W1n_ref[...], b1n_ref[...],
                            W2n_ref[...], b2n_ref[...])
    P = x @ W1et_ref[...]
    sv = jnp.sum(x, axis=-1, keepdims=True)
    sqv = jnp.sum(x * x, axis=-1, keepdims=True)
    pad = jnp.zeros((x.shape[0], TVW - D - 2), jnp.float32)
    tv_ref[...] = jnp.concatenate((P, sv, sqv, pad), axis=-1)


def _stage_a(v, W1n, b1n, W2n, b2n, W1e):
    return pl.pallas_call(
        _stage_a_kernel,
        grid=(N // TILE_N,),
        in_specs=[
            pl.BlockSpec((TILE_N, D), lambda i: (i, 0)),
            pl.BlockSpec((D, D), lambda i: (0, 0)),
            pl.BlockSpec((D,), lambda i: (0,)),
            pl.BlockSpec((D, D), lambda i: (0, 0)),
            pl.BlockSpec((D,), lambda i: (0,)),
            pl.BlockSpec((D, D), lambda i: (0, 0)),
        ],
        out_specs=[
            pl.BlockSpec((TILE_N, D), lambda i: (i, 0)),
            pl.BlockSpec((TILE_N, TVW), lambda i: (i, 0)),
        ],
        out_shape=[
            jax.ShapeDtypeStruct((N, D), jnp.float32),
            jax.ShapeDtypeStruct((N, TVW), jnp.float32),
        ],
    )(v, W1n, b1n, W2n, b2n, W1e[:D])


def _stage_c_kernel(es_ref, e0_ref, ers_ref, e_ref, W1eb_ref, ab_ref,
                    edge_ref, te_ref):
    alpha = ab_ref[0]
    es = es_ref[...]
    s = es[:, :D]
    cnt = es[:, D:D + 1]
    mean = s / jnp.maximum(cnt, 1.0)
    edge = ((1.0 - alpha) * mean + alpha * e0_ref[...]) / ers_ref[...]
    edge_ref[...] = edge
    Q = edge @ W1eb_ref[...]
    se = jnp.sum(edge, axis=-1, keepdims=True)
    sqe = jnp.sum(edge * edge, axis=-1, keepdims=True)
    pad = jnp.zeros((edge.shape[0], TEW - 2 * D - 2), jnp.float32)
    te_ref[...] = jnp.concatenate((Q, e_ref[...], se, sqe, pad), axis=-1)


def _stage_c(es_pack, e0, e_reg_sum, e, W1e, alpha):
    return pl.pallas_call(
        _stage_c_kernel,
        grid=(M // TILE_M,),
        in_specs=[
            pl.BlockSpec((TILE_M, A1W), lambda i: (i, 0)),
            pl.BlockSpec((TILE_M, D), lambda i: (i, 0)),
            pl.BlockSpec((TILE_M, 1), lambda i: (i, 0)),
            pl.BlockSpec((TILE_M, D), lambda i: (i, 0)),
            pl.BlockSpec((D, D), lambda i: (0, 0)),
            pl.BlockSpec((1,), lambda i: (0,)),
        ],
        out_specs=[
            pl.BlockSpec((TILE_M, D), lambda i: (i, 0)),
            pl.BlockSpec((TILE_M, TEW), lambda i: (i, 0)),
        ],
        out_shape=[
            jax.ShapeDtypeStruct((M, D), jnp.float32),
            jax.ShapeDtypeStruct((M, TEW), jnp.float32),
        ],
    )(es_pack, e0, e_reg_sum,
      e, W1e[D:], jnp.full((1,), alpha, jnp.float32))


def _stage_e_kernel(tvg_ref, teg_ref, erw_ref, c_ref, b1e_ref,
                    a1_ref, a2_ref):
    tvg = tvg_ref[...]
    teg = teg_ref[...]
    P = tvg[:, :D]
    sv = tvg[:, D:D + 1]
    sqv = tvg[:, D + 1:D + 2]
    Q = teg[:, :D]
    eg = teg[:, D:2 * D]
    se = teg[:, 2 * D:2 * D + 1]
    sqe = teg[:, 2 * D + 1:2 * D + 2]
    mu = (sv + se) * (1.0 / 256.0)
    var = (sqv + sqe) * (1.0 / 256.0) - mu * mu
    inv = jax.lax.rsqrt(var + 1e-5)
    g = (P + Q - mu * c_ref[...]) * inv + b1e_ref[...]
    m2 = jnp.mean(g, axis=-1, keepdims=True)
    v2 = jnp.mean(g * g, axis=-1, keepdims=True) - m2 * m2
    h = jax.nn.relu((g - m2) * jax.lax.rsqrt(v2 + 1e-5))
    erw = erw_ref[...]
    u = erw * h
    ones = jnp.ones((u.shape[0], 1), jnp.float32)
    pad = jnp.zeros((u.shape[0], A1W - D - 2), jnp.float32)
    a1_ref[...] = jnp.concatenate((u, erw, ones, pad), axis=-1)
    a2_ref[...] = erw * eg


def _stage_e(tvg, teg, erw, csum, b1e):
    return pl.pallas_call(
        _stage_e_kernel,
        grid=(E // TILE_E,),
        in_specs=[
            pl.BlockSpec((TILE_E, TVW), lambda i: (i, 0)),
            pl.BlockSpec((TILE_E, TEW), lambda i: (i, 0)),
            pl.BlockSpec((TILE_E, 1), lambda i: (i, 0)),
            pl.BlockSpec((D,), lambda i: (0,)),
            pl.BlockSpec((D,), lambda i: (0,)),
        ],
        out_specs=[
            pl.BlockSpec((TILE_E, A1W), lambda i: (i, 0)),
            pl.BlockSpec((TILE_E, D), lambda i: (i, 0)),
        ],
        out_shape=[
            jax.ShapeDtypeStruct((E, A1W), jnp.float32),
            jax.ShapeDtypeStruct((E, D), jnp.float32),
        ],
    )(tvg, teg, erw, csum, b1e)


def _stage_d_kernel(uc_ref, wv_ref, nrs_ref, v0_ref, W2e_ref, b2e_ref,
                    W1a_ref, b1a_ref, W2a_ref, b2a_ref, ab_ref, out_ref):
    alpha = ab_ref[0]
    beta = ab_ref[1]
    uc = uc_ref[...]
    U = uc[:, :D]
    S = uc[:, D:D + 1]
    C = uc[:, D + 1:D + 2]
    ns = beta * (U @ W2e_ref[...]) + (beta * S) * b2e_ref[...] \
        + (1.0 - beta) * wv_ref[...]
    node = ns / jnp.maximum(C, 1.0)
    node = node / nrs_ref[...]
    node = (1.0 - alpha) * node + alpha * v0_ref[...]
    out = _mlp_body(_ln(node), W1a_ref[...], b1a_ref[...],
                    W2a_ref[...], b2a_ref[...])
    out_ref[...] = beta * out + (1.0 - beta) * node


def _stage_d(uc, wv, n_reg_sum, v0, W2e, b2e, W1a, b1a, W2a, b2a, alpha, beta):
    ab = jnp.stack([jnp.float32(alpha), jnp.float32(beta)])
    return pl.pallas_call(
        _stage_d_kernel,
        grid=(N // TILE_N,),
        in_specs=[
            pl.BlockSpec((TILE_N, A1W), lambda i: (i, 0)),
            pl.BlockSpec((TILE_N, D), lambda i: (i, 0)),
            pl.BlockSpec((TILE_N, 1), lambda i: (i, 0)),
            pl.BlockSpec((TILE_N, D), lambda i: (i, 0)),
            pl.BlockSpec((D, D), lambda i: (0, 0)),
            pl.BlockSpec((D,), lambda i: (0,)),
            pl.BlockSpec((D, D), lambda i: (0, 0)),
            pl.BlockSpec((D,), lambda i: (0,)),
            pl.BlockSpec((D, D), lambda i: (0, 0)),
            pl.BlockSpec((D,), lambda i: (0,)),
            pl.BlockSpec((2,), lambda i: (0,)),
        ],
        out_specs=pl.BlockSpec((TILE_N, D), lambda i: (i, 0)),
        out_shape=jax.ShapeDtypeStruct((N, D), jnp.float32),
    )(uc, wv, n_reg_sum, v0, W2e, b2e, W1a, b1a, W2a, b2a, ab)


def kernel(v, e, v0, e0, n_reg_weight, e_reg_weight, n_reg_sum, e_reg_sum,
           W1n, b1n, W2n, b2n, W1e, b1e, W2e, b2e, W1a, b1a, W2a, b2a,
           vidx, eidx, alpha, beta):
    node_msg, tv = _stage_a(v, W1n, b1n, W2n, b2n, W1e)

    g1 = node_msg[vidx] * n_reg_weight
    pay1 = jnp.concatenate(
        (g1, jnp.ones((E, 1), jnp.float32), jnp.zeros((E, A1W - D - 1), jnp.float32)),
        axis=-1)
    es_pack = jax.ops.segment_sum(pay1, eidx, num_segments=M)

    csum = jnp.sum(W1e, axis=0)
    edge, te = _stage_c(es_pack, e0, e_reg_sum, e, W1e, alpha)

    tvg = tv[vidx]
    teg = te[eidx]
    a1, a2 = _stage_e(tvg, teg, e_reg_weight, csum, b1e)

    uc = jax.ops.segment_sum(a1, vidx, num_segments=N)
    wv = jax.ops.segment_sum(a2, vidx, num_segments=N)

    node = _stage_d(uc, wv, n_reg_sum, v0, W2e, b2e, W1a, b1a, W2a, b2a,
                    alpha, beta)
    return (node, edge)


# speedup vs baseline: 1.4827x; 1.4827x over previous
import functools

import jax
import jax.numpy as jnp
from jax import lax
from jax.experimental import pallas as pl
from jax.experimental.pallas import tpu as pltpu
from jax.experimental.pallas import tpu_sc as plsc

N = 10000
M = 5000
E = 320000
D = 128

CH = 128
NCH = 2560
E_PAD = NCH * CH
N_ACC = 10240
M_ACC = 5120
WA = 128
WB = 256
WE = 384
PW = 128
CW = 16

TILE_N = 1000
TILE_M = 1000
TILE_EP = 2048

_mesh = plsc.VectorSubcoreMesh(core_axis_name="c", subcore_axis_name="s")


def _gather2_body(ta_hbm, tb_hbm, idx_hbm,
                  outa_hbm, outb_hbm,
                  idxbuf, bufa, bufb, sem):
    c = lax.axis_index("c")
    s = lax.axis_index("s")
    wid = c * 16 + s

    @pl.loop(0, NCH // 32)
    def _(t):
        ch = wid * (NCH // 32) + t
        pltpu.sync_copy(idx_hbm.at[ch], idxbuf)
        pltpu.async_copy(ta_hbm.at[idxbuf], bufa, sem).wait()
        pltpu.async_copy(tb_hbm.at[idxbuf], bufb, sem).wait()
        pltpu.sync_copy(bufa, outa_hbm.at[pl.ds(ch * CH, CH)])
        pltpu.sync_copy(bufb, outb_hbm.at[pl.ds(ch * CH, CH)])


def _gather2(ta, tb, idx2d):
    k = functools.partial(
        pl.kernel,
        out_type=[
            jax.ShapeDtypeStruct((E_PAD, WA), jnp.float32),
            jax.ShapeDtypeStruct((E_PAD, WB), jnp.float32),
        ],
        mesh=_mesh,
        scratch_types=[
            pltpu.VMEM((CH,), jnp.int32),
            pltpu.VMEM((CH, WA), jnp.float32),
            pltpu.VMEM((CH, WB), jnp.float32),
            pltpu.SemaphoreType.DMA,
        ],
    )(_gather2_body)
    return k(ta, tb, idx2d)


def _gather_e_body(te_hbm, idx_hbm, out_hbm, idxbuf, buf, sem):
    c = lax.axis_index("c")
    s = lax.axis_index("s")
    wid = c * 16 + s

    @pl.loop(0, NCH // 32)
    def _(t):
        ch = wid * (NCH // 32) + t
        pltpu.sync_copy(idx_hbm.at[ch], idxbuf)
        pltpu.async_copy(te_hbm.at[idxbuf], buf, sem).wait()
        pltpu.sync_copy(buf, out_hbm.at[pl.ds(ch * CH, CH)])


def _gather_e(te, idx2d):
    k = functools.partial(
        pl.kernel,
        out_type=jax.ShapeDtypeStruct((E_PAD, WE), jnp.float32),
        mesh=_mesh,
        scratch_types=[
            pltpu.VMEM((CH,), jnp.int32),
            pltpu.VMEM((CH, WE), jnp.float32),
            pltpu.SemaphoreType.DMA,
        ],
    )(_gather_e_body)
    return k(te, idx2d)


def _scat_e_body(pay_hbm, idx_hbm, zrs_hbm, ones_hbm,
                 outv_hbm, outc_hbm,
                 idxbuf, paybuf, onesbuf, accv, accc):
    c = lax.axis_index("c")
    s = lax.axis_index("s")
    stripe = M_ACC // 16
    pltpu.sync_copy(zrs_hbm, paybuf)
    for off, sz in ((0, 128), (128, 128), (256, 64)):
        pltpu.sync_copy(paybuf.at[pl.ds(0, sz)],
                        accv.at[pl.ds(s * stripe + off, sz)])
        pltpu.sync_copy(paybuf.at[pl.ds(0, sz)],
                        accc.at[pl.ds(s * stripe + off, sz)])
    pltpu.sync_copy(ones_hbm, onesbuf)
    plsc.subcore_barrier()

    @pl.loop(0, NCH // 32)
    def _(t):
        ch = c * (NCH // 2) + s * (NCH // 32) + t
        pltpu.sync_copy(idx_hbm.at[ch], idxbuf)
        pltpu.sync_copy(pay_hbm.at[pl.ds(ch * CH, CH)], paybuf)
        pltpu.sync_copy(paybuf, accv.at[idxbuf], add=True)
        pltpu.sync_copy(onesbuf, accc.at[idxbuf], add=True)

    plsc.subcore_barrier()
    for off, sz in ((0, 128), (128, 128), (256, 64)):
        pltpu.sync_copy(accv.at[pl.ds(s * stripe + off, sz)],
                        paybuf.at[pl.ds(0, sz)])
        pltpu.sync_copy(paybuf.at[pl.ds(0, sz)],
                        outv_hbm.at[c, pl.ds(s * stripe + off, sz)])
        pltpu.sync_copy(accc.at[pl.ds(s * stripe + off, sz)],
                        onesbuf.at[pl.ds(0, sz)])
        pltpu.sync_copy(onesbuf.at[pl.ds(0, sz)],
                        outc_hbm.at[c, pl.ds(s * stripe + off, sz)])


def _scat_e(pay, idx2d, zrs, ones):
    k = functools.partial(
        pl.kernel,
        out_type=[
            jax.ShapeDtypeStruct((2, M_ACC, PW), jnp.float32),
            jax.ShapeDtypeStruct((2, M_ACC, PW), jnp.float32),
        ],
        mesh=_mesh,
        scratch_types=[
            pltpu.VMEM((CH,), jnp.int32),
            pltpu.VMEM((CH, PW), jnp.float32),
            pltpu.VMEM((CH, PW), jnp.float32),
            pltpu.VMEM_SHARED((M_ACC, PW), jnp.float32),
            pltpu.VMEM_SHARED((M_ACC, PW), jnp.float32),
        ],
    )(_scat_e_body)
    return k(pay, idx2d, zrs, ones)


def _cnt_n_body(idx_hbm, zrs_hbm, ones_hbm, out_hbm, idxbuf, onesbuf, acc):
    c = lax.axis_index("c")
    s = lax.axis_index("s")
    stripe = N_ACC // 16
    pltpu.sync_copy(zrs_hbm, onesbuf)
    for off in range(0, stripe, 128):
        pltpu.sync_copy(onesbuf,
                        acc.at[pl.ds(s * stripe + off, 128)])
    pltpu.sync_copy(ones_hbm, onesbuf)
    plsc.subcore_barrier()

    @pl.loop(0, NCH // 32)
    def _(t):
        ch = c * (NCH // 2) + s * (NCH // 32) + t
        pltpu.sync_copy(idx_hbm.at[ch], idxbuf)
        pltpu.sync_copy(onesbuf, acc.at[idxbuf], add=True)

    plsc.subcore_barrier()
    for off in range(0, stripe, 128):
        pltpu.sync_copy(acc.at[pl.ds(s * stripe + off, 128)], onesbuf)
        pltpu.sync_copy(onesbuf,
                        out_hbm.at[c, pl.ds(s * stripe + off, 128)])


def _cnt_n(idx2d, zrs, ones):
    k = functools.partial(
        pl.kernel,
        out_type=jax.ShapeDtypeStruct((2, N_ACC, PW), jnp.float32),
        mesh=_mesh,
        scratch_types=[
            pltpu.VMEM((CH,), jnp.int32),
            pltpu.VMEM((CH, PW), jnp.float32),
            pltpu.VMEM_SHARED((N_ACC, PW), jnp.float32),
        ],
    )(_cnt_n_body)
    return k(idx2d, zrs, ones)


def _scat_n_body(pay_hbm, idx_hbm, zrs_hbm, out_hbm, idxbuf, paybuf, acc):
    c = lax.axis_index("c")
    s = lax.axis_index("s")
    stripe = N_ACC // 16
    pltpu.sync_copy(zrs_hbm, paybuf)
    for off in range(0, stripe, 128):
        pltpu.sync_copy(paybuf,
                        acc.at[pl.ds(s * stripe + off, 128)])
    plsc.subcore_barrier()

    @pl.loop(0, NCH // 32)
    def _(t):
        ch = c * (NCH // 2) + s * (NCH // 32) + t
        pltpu.sync_copy(idx_hbm.at[ch], idxbuf)
        pltpu.sync_copy(pay_hbm.at[pl.ds(ch * CH, CH)], paybuf)
        pltpu.sync_copy(paybuf, acc.at[idxbuf], add=True)

    plsc.subcore_barrier()
    for off in range(0, stripe, 128):
        pltpu.sync_copy(acc.at[pl.ds(s * stripe + off, 128)], paybuf)
        pltpu.sync_copy(paybuf,
                        out_hbm.at[c, pl.ds(s * stripe + off, 128)])


def _scat_n(pay, idx2d, zrs):
    k = functools.partial(
        pl.kernel,
        out_type=jax.ShapeDtypeStruct((2, N_ACC, PW), jnp.float32),
        mesh=_mesh,
        scratch_types=[
            pltpu.VMEM((CH,), jnp.int32),
            pltpu.VMEM((CH, PW), jnp.float32),
            pltpu.VMEM_SHARED((N_ACC, PW), jnp.float32),
        ],
    )(_scat_n_body)
    return k(pay, idx2d, zrs)


def _ln(x, eps=1e-5):
    mu = jnp.mean(x, axis=-1, keepdims=True)
    var = jnp.var(x, axis=-1, keepdims=True)
    return (x - mu) / jnp.sqrt(var + eps)


def _mlp_body(x, W1, b1, W2, b2):
    h = jax.nn.relu(_ln(x @ W1 + b1))
    return h @ W2 + b2


def _stage_a_kernel(v_ref, W1n_ref, b1n_ref, W2n_ref, b2n_ref, W1et_ref,
                    ta_ref, tb_ref):
    x = v_ref[...]
    nm = _mlp_body(_ln(x), W1n_ref[...], b1n_ref[...], W2n_ref[...],
                   b2n_ref[...])
    ta_ref[...] = nm
    P = x @ W1et_ref[...]
    sv = jnp.sum(x, axis=-1, keepdims=True)
    sqv = jnp.sum(x * x, axis=-1, keepdims=True)
    pad = jnp.zeros((x.shape[0], WB - D - 2), jnp.float32)
    tb_ref[...] = jnp.concatenate((P, sv, sqv, pad), axis=-1)


def _stage_a(v, W1n, b1n, W2n, b2n, W1e):
    return pl.pallas_call(
        _stage_a_kernel,
        grid=(N // TILE_N,),
        in_specs=[
            pl.BlockSpec((TILE_N, D), lambda i: (i, 0)),
            pl.BlockSpec((D, D), lambda i: (0, 0)),
            pl.BlockSpec((D,), lambda i: (0,)),
            pl.BlockSpec((D, D), lambda i: (0, 0)),
            pl.BlockSpec((D,), lambda i: (0,)),
            pl.BlockSpec((D, D), lambda i: (0, 0)),
        ],
        out_specs=[
            pl.BlockSpec((TILE_N, WA), lambda i: (i, 0)),
            pl.BlockSpec((TILE_N, WB), lambda i: (i, 0)),
        ],
        out_shape=[
            jax.ShapeDtypeStruct((N, WA), jnp.float32),
            jax.ShapeDtypeStruct((N, WB), jnp.float32),
        ],
    )(v, W1n, b1n, W2n, b2n, W1e[:D])


def _pay1_kernel(ga_ref, nrw_ref, out_ref):
    out_ref[...] = ga_ref[...] * nrw_ref[...]


def _pay1(ga, nrw_p):
    return pl.pallas_call(
        _pay1_kernel,
        grid=(E_PAD // TILE_EP,),
        in_specs=[
            pl.BlockSpec((TILE_EP, WA), lambda i: (i, 0)),
            pl.BlockSpec((TILE_EP, 1), lambda i: (i, 0)),
        ],
        out_specs=pl.BlockSpec((TILE_EP, PW), lambda i: (i, 0)),
        out_shape=jax.ShapeDtypeStruct((E_PAD, PW), jnp.float32),
    )(ga, nrw_p)


def _stage_c_kernel(s0_ref, s1_ref, c0_ref, c1_ref, e0_ref, ers_ref, e_ref,
                    W1eb_ref, ab_ref, edge_ref, te_ref):
    alpha = ab_ref[0]
    s = s0_ref[...] + s1_ref[...]
    cnt = c0_ref[...] + c1_ref[...]
    mean = s / jnp.maximum(cnt, 1.0)
    edge = ((1.0 - alpha) * mean + alpha * e0_ref[...]) / ers_ref[...]
    edge_ref[...] = edge
    Q = edge @ W1eb_ref[...]
    se = jnp.sum(edge, axis=-1, keepdims=True)
    sqe = jnp.sum(edge * edge, axis=-1, keepdims=True)
    pad = jnp.zeros((edge.shape[0], WE - 2 * D - 2), jnp.float32)
    te_ref[...] = jnp.concatenate((Q, e_ref[...], se, sqe, pad), axis=-1)


def _stage_c(s0, s1, c0, c1, e0, e_reg_sum, e, W1e, alpha):
    return pl.pallas_call(
        _stage_c_kernel,
        grid=(M // TILE_M,),
        in_specs=[
            pl.BlockSpec((TILE_M, PW), lambda i: (i, 0)),
            pl.BlockSpec((TILE_M, PW), lambda i: (i, 0)),
            pl.BlockSpec((TILE_M, 1), lambda i: (i, 0)),
            pl.BlockSpec((TILE_M, 1), lambda i: (i, 0)),
            pl.BlockSpec((TILE_M, D), lambda i: (i, 0)),
            pl.BlockSpec((TILE_M, 1), lambda i: (i, 0)),
            pl.BlockSpec((TILE_M, D), lambda i: (i, 0)),
            pl.BlockSpec((D, D), lambda i: (0, 0)),
            pl.BlockSpec((1,), lambda i: (0,)),
        ],
        out_specs=[
            pl.BlockSpec((TILE_M, D), lambda i: (i, 0)),
            pl.BlockSpec((TILE_M, WE), lambda i: (i, 0)),
        ],
        out_shape=[
            jax.ShapeDtypeStruct((M, D), jnp.float32),
            jax.ShapeDtypeStruct((M, WE), jnp.float32),
        ],
    )(s0, s1, c0, c1, e0, e_reg_sum, e, W1e[D:],
      jnp.full((1,), alpha, jnp.float32))


def _stage_e_kernel(gb_ref, ge_ref, erw_ref, csum_ref, b1e_ref, W2e_ref,
                    b2e_ref, ab_ref, out_ref):
    beta = ab_ref[1]
    gb = gb_ref[...]
    ge = ge_ref[...]
    P = gb[:, :D]
    sv = gb[:, D:D + 1]
    sqv = gb[:, D + 1:D + 2]
    Q = ge[:, :D]
    eg = ge[:, D:2 * D]
    se = ge[:, 2 * D:2 * D + 1]
    sqe = ge[:, 2 * D + 1:2 * D + 2]
    mu = (sv + se) * (1.0 / 256.0)
    var = (sqv + sqe) * (1.0 / 256.0) - mu * mu
    inv = lax.rsqrt(var + 1e-5)
    g = (P + Q - mu * csum_ref[...]) * inv + b1e_ref[...]
    m2 = jnp.mean(g, axis=-1, keepdims=True)
    v2 = jnp.mean(g * g, axis=-1, keepdims=True) - m2 * m2
    h = jax.nn.relu((g - m2) * lax.rsqrt(v2 + 1e-5))
    msg = beta * (h @ W2e_ref[...] + b2e_ref[...]) + (1.0 - beta) * eg
    out_ref[...] = erw_ref[...] * msg


def _stage_e(gb, ge, erw_p, csum, b1e, W2e, b2e, alpha, beta):
    ab = jnp.stack([jnp.float32(alpha), jnp.float32(beta)])
    return pl.pallas_call(
        _stage_e_kernel,
        grid=(E_PAD // TILE_EP,),
        in_specs=[
            pl.BlockSpec((TILE_EP, WB), lambda i: (i, 0)),
            pl.BlockSpec((TILE_EP, WE), lambda i: (i, 0)),
            pl.BlockSpec((TILE_EP, 1), lambda i: (i, 0)),
            pl.BlockSpec((D,), lambda i: (0,)),
            pl.BlockSpec((D,), lambda i: (0,)),
            pl.BlockSpec((D, D), lambda i: (0, 0)),
            pl.BlockSpec((D,), lambda i: (0,)),
            pl.BlockSpec((2,), lambda i: (0,)),
        ],
        out_specs=pl.BlockSpec((TILE_EP, PW), lambda i: (i, 0)),
        out_shape=jax.ShapeDtypeStruct((E_PAD, PW), jnp.float32),
    )(gb, ge, erw_p, csum, b1e, W2e, b2e, ab)


def _stage_d_kernel(u0_ref, u1_ref, c0_ref, c1_ref, nrs_ref, v0_ref,
                    W1a_ref, b1a_ref, W2a_ref, b2a_ref, ab_ref, out_ref):
    alpha = ab_ref[0]
    beta = ab_ref[1]
    U = u0_ref[...] + u1_ref[...]
    C = c0_ref[...] + c1_ref[...]
    node = U / jnp.maximum(C, 1.0)
    node = node / nrs_ref[...]
    node = (1.0 - alpha) * node + alpha * v0_ref[...]
    out = _mlp_body(_ln(node), W1a_ref[...], b1a_ref[...], W2a_ref[...],
                    b2a_ref[...])
    out_ref[...] = beta * out + (1.0 - beta) * node


def _stage_d(u0, u1, c0, c1, n_reg_sum, v0, W1a, b1a, W2a, b2a, alpha, beta):
    ab = jnp.stack([jnp.float32(alpha), jnp.float32(beta)])
    return pl.pallas_call(
        _stage_d_kernel,
        grid=(N // TILE_N,),
        in_specs=[
            pl.BlockSpec((TILE_N, PW), lambda i: (i, 0)),
            pl.BlockSpec((TILE_N, PW), lambda i: (i, 0)),
            pl.BlockSpec((TILE_N, 1), lambda i: (i, 0)),
            pl.BlockSpec((TILE_N, 1), lambda i: (i, 0)),
            pl.BlockSpec((TILE_N, 1), lambda i: (i, 0)),
            pl.BlockSpec((TILE_N, D), lambda i: (i, 0)),
            pl.BlockSpec((D, D), lambda i: (0, 0)),
            pl.BlockSpec((D,), lambda i: (0,)),
            pl.BlockSpec((D, D), lambda i: (0, 0)),
            pl.BlockSpec((D,), lambda i: (0,)),
            pl.BlockSpec((2,), lambda i: (0,)),
        ],
        out_specs=pl.BlockSpec((TILE_N, D), lambda i: (i, 0)),
        out_shape=jax.ShapeDtypeStruct((N, D), jnp.float32),
    )(u0, u1, c0, c1, n_reg_sum, v0, W1a, b1a, W2a, b2a, ab)


def kernel(v, e, v0, e0, n_reg_weight, e_reg_weight, n_reg_sum, e_reg_sum,
           W1n, b1n, W2n, b2n, W1e, b1e, W2e, b2e, W1a, b1a, W2a, b2a,
           vidx, eidx, alpha, beta):
    npad = E_PAD - E
    vidx2d = jnp.concatenate(
        (vidx.astype(jnp.int32), jnp.full((npad,), N, jnp.int32))
    ).reshape(NCH, CH)
    eidx2d = jnp.concatenate(
        (eidx.astype(jnp.int32), jnp.full((npad,), M, jnp.int32))
    ).reshape(NCH, CH)
    nrw_p = jnp.concatenate((n_reg_weight, jnp.zeros((npad, 1), jnp.float32)))
    erw_p = jnp.concatenate((e_reg_weight, jnp.zeros((npad, 1), jnp.float32)))
    zrs = jnp.zeros((128, PW), jnp.float32)
    ones128 = jnp.ones((CH, PW), jnp.float32)

    ta, tb = _stage_a(v, W1n, b1n, W2n, b2n, W1e)
    ta_p = jnp.pad(ta, ((0, 16), (0, 0)))
    tb_p = jnp.pad(tb, ((0, 16), (0, 0)))

    ga, gb = _gather2(ta_p, tb_p, vidx2d)
    cn = _cnt_n(vidx2d, zrs, ones128)
    p1 = _pay1(ga, nrw_p)
    es, ec = _scat_e(p1, eidx2d, zrs, ones128)

    csum = jnp.sum(W1e, axis=0)
    edge, te = _stage_c(es[0, :M], es[1, :M], ec[0, :M, :1], ec[1, :M, :1],
                        e0, e_reg_sum, e, W1e, alpha)
    te_p = jnp.pad(te, ((0, 8), (0, 0)))

    ge = _gather_e(te_p, eidx2d)
    u = _stage_e(gb, ge, erw_p, csum, b1e, W2e, b2e, alpha, beta)
    un = _scat_n(u, vidx2d, zrs)

    node = _stage_d(un[0, :N], un[1, :N], cn[0, :N, :1], cn[1, :N, :1],
                    n_reg_sum, v0, W1a, b1a, W2a, b2a, alpha, beta)
    return (node, edge)
